# baseline (speedup 1.0000x reference)
import functools

import jax
import jax.numpy as jnp
import numpy as np
from jax import lax
from jax.experimental import pallas as pl
from jax.experimental.pallas import tpu as pltpu
from jax.experimental.pallas import tpu_sc as plsc

N = 16777216
NC, NS, L = 2, 16, 16
NW = NC * NS
UNIT = 524288
K_TC = 10
M = K_TC * UNIT
TAIL = N - M
PER_W = TAIL // NW
CHUNK = 16384
NCHUNK = PER_W // CHUNK
UNROLL = 8

COLS = 1024
BM = 512

_SCALE = np.float32(8.0 / 7.0)
_BIAS = np.float32(64.0 / 7.0)


def _tables():
    seg_fp = np.linspace(-8.0, 6.0, 17).astype(np.float32)
    silu_fp = (seg_fp.astype(np.float64) /
               (1.0 + np.exp(-seg_fp.astype(np.float64))))
    silu_vals = np.round(silu_fp * 1024.0).astype(np.int64)
    y0 = silu_vals[:16] / 1024.0
    dy = (silu_vals[1:] - silu_vals[:16]) / 1024.0
    a = (y0 - np.arange(16) * dy).astype(np.float32)
    return a, dy.astype(np.float32)


_Y0, _DY = _tables()
_Y0_PAD = np.zeros(128, dtype=np.float32)
_Y0_PAD[:16] = _Y0
_DY_PAD = np.zeros(128, dtype=np.float32)
_DY_PAD[:16] = _DY

_GATHER_DNUMS = lax.GatherDimensionNumbers(
    offset_dims=(), collapsed_slice_dims=(0,), start_index_map=(0,))


def _vgather(table, idx):
    return lax.gather(
        table, idx[:, None], _GATHER_DNUMS, (1,),
        mode=lax.GatherScatterMode.PROMISE_IN_BOUNDS)


def _silu_vec(xv, a_t, dy_t):
    v = jnp.maximum(xv * _SCALE + _BIAS, 0.0)
    idx = jnp.minimum(v, 15.5).astype(jnp.int32)
    return _vgather(a_t, idx) + v * _vgather(dy_t, idx)


def _compute_chunk(xb, ob, a_t, dy_t):
    def vec_step(k, _):
        for u in range(UNROLL):
            j = (k * UNROLL + u) * L
            ob[pl.ds(j, L)] = _silu_vec(xb[pl.ds(j, L)], a_t, dy_t)
        return 0

    lax.fori_loop(0, CHUNK // (L * UNROLL), vec_step, 0)


def _sc_body(x_hbm, y0_hbm, dy_hbm, out_hbm,
             xb0, xb1, ob0, ob1, y0_v, dy_v, is0, is1, os0, os1):
    wid = lax.axis_index("s") * NC + lax.axis_index("c")
    base = M + wid * PER_W
    pltpu.sync_copy(y0_hbm, y0_v)
    pltpu.sync_copy(dy_hbm, dy_v)
    a_t = y0_v[...]
    dy_t = dy_v[...]

    pltpu.async_copy(x_hbm.at[pl.ds(base, CHUNK)], xb0, is0)
    bufs = ((xb0, ob0, is0, os0), (xb1, ob1, is1, os1))

    def step(ci2, _):
        i0 = ci2 * 2
        for ph in (0, 1):
            xb, ob, isem, osem = bufs[ph]
            nxb, _, nisem, _ = bufs[1 - ph]
            i = i0 + ph
            nxt = i + 1

            if ph == 0:
                pltpu.async_copy(
                    x_hbm.at[pl.ds(base + nxt * CHUNK, CHUNK)], nxb, nisem)
            else:
                @pl.when(nxt < NCHUNK)
                def _():
                    pltpu.async_copy(
                        x_hbm.at[pl.ds(base + nxt * CHUNK, CHUNK)], nxb, nisem)

            pltpu.make_async_copy(
                x_hbm.at[pl.ds(base, CHUNK)], xb, isem).wait()

            @pl.when(i >= 2)
            def _():
                pltpu.make_async_copy(
                    ob, out_hbm.at[pl.ds(base, CHUNK)], osem).wait()

            _compute_chunk(xb, ob, a_t, dy_t)
            pltpu.async_copy(
                ob, out_hbm.at[pl.ds(base + i * CHUNK, CHUNK)], osem)
        return 0

    lax.fori_loop(0, NCHUNK // 2, step, 0)
    pltpu.make_async_copy(ob0, out_hbm.at[pl.ds(base, CHUNK)], os0).wait()
    pltpu.make_async_copy(ob1, out_hbm.at[pl.ds(base, CHUNK)], os1).wait()


@functools.partial(
    pl.kernel,
    mesh=plsc.VectorSubcoreMesh(core_axis_name="c", subcore_axis_name="s"),
    out_type=jax.ShapeDtypeStruct((N,), jnp.float32),
    scratch_types=[
        pltpu.VMEM((CHUNK,), jnp.float32),
        pltpu.VMEM((CHUNK,), jnp.float32),
        pltpu.VMEM((CHUNK,), jnp.float32),
        pltpu.VMEM((CHUNK,), jnp.float32),
        pltpu.VMEM((16,), jnp.float32),
        pltpu.VMEM((16,), jnp.float32),
        pltpu.SemaphoreType.DMA,
        pltpu.SemaphoreType.DMA,
        pltpu.SemaphoreType.DMA,
        pltpu.SemaphoreType.DMA,
    ],
)
def _silu_sc(x_hbm, y0_hbm, dy_hbm, out_hbm,
             xb0, xb1, ob0, ob1, y0_v, dy_v, is0, is1, os0, os1):
    _sc_body(x_hbm, y0_hbm, dy_hbm, out_hbm,
             xb0, xb1, ob0, ob1, y0_v, dy_v, is0, is1, os0, os1)


def _tc_head_body(a_ref, d_ref, x_ref, o_ref):
    xv = x_ref[...]
    v = jnp.maximum(xv * _SCALE + _BIAS, 0.0)
    idx = jnp.minimum(v, 15.5).astype(jnp.int32)
    at = jnp.broadcast_to(a_ref[...], (BM, 128))
    dt = jnp.broadcast_to(d_ref[...], (BM, 128))
    a = jnp.take_along_axis(at, idx, axis=1)
    dy = jnp.take_along_axis(dt, idx, axis=1)
    o_ref[...] = a + v * dy


def _tc_head(x_head2d):
    return pl.pallas_call(
        _tc_head_body,
        out_shape=jax.ShapeDtypeStruct((M // COLS, COLS), jnp.float32),
        grid=(M // (BM * COLS),),
        in_specs=[
            pl.BlockSpec((1, 128), lambda i: (0, 0)),
            pl.BlockSpec((1, 128), lambda i: (0, 0)),
            pl.BlockSpec((BM, COLS), lambda i: (i, 0)),
        ],
        out_specs=pl.BlockSpec((BM, COLS), lambda i: (i, 0)),
    )(jnp.asarray(_Y0_PAD)[None], jnp.asarray(_DY_PAD)[None], x_head2d)


def _stitch_body(full_ref, head_ref, o_ref):
    del full_ref
    o_ref[...] = head_ref[...]


def _stitch(full, head2d):
    full2d = full.reshape(N // COLS, COLS)
    out = pl.pallas_call(
        _stitch_body,
        out_shape=jax.ShapeDtypeStruct((N // COLS, COLS), jnp.float32),
        grid=(M // (BM * COLS),),
        in_specs=[
            pl.BlockSpec(memory_space=pl.ANY),
            pl.BlockSpec((BM, COLS), lambda i: (i, 0)),
        ],
        out_specs=pl.BlockSpec((BM, COLS), lambda i: (i, 0)),
        input_output_aliases={0: 0},
    )(full2d, head2d)
    return out.reshape(N)


def kernel(x):
    head2d = x[:M].reshape(M // COLS, COLS)
    out_head = _tc_head(head2d)
    full = _silu_sc(x, jnp.asarray(_Y0), jnp.asarray(_DY))
    return _stitch(full, out_head)


# speedup vs baseline: 2.3348x; 2.3348x over previous
import functools

import jax
import jax.numpy as jnp
import numpy as np
from jax import lax
from jax.experimental import pallas as pl
from jax.experimental.pallas import tpu as pltpu
from jax.experimental.pallas import tpu_sc as plsc

N = 16777216
NC, NS, L = 2, 16, 16
NW = NC * NS
PER_W = N // NW
CHUNK = 16384
NCHUNK = PER_W // CHUNK
UNROLL = 8

_SCALE = np.float32(8.0 / 7.0)
_BIAS = np.float32(64.0 / 7.0)


def _tables():
    seg_fp = np.linspace(-8.0, 6.0, 17).astype(np.float32)
    silu_fp = (seg_fp.astype(np.float64) /
               (1.0 + np.exp(-seg_fp.astype(np.float64))))
    silu_vals = np.round(silu_fp * 1024.0).astype(np.int64)
    y0 = silu_vals[:16] / 1024.0
    dy = (silu_vals[1:] - silu_vals[:16]) / 1024.0
    a = (y0 - np.arange(16) * dy).astype(np.float32)
    return a, dy.astype(np.float32)


_Y0, _DY = _tables()


_GATHER_DNUMS = lax.GatherDimensionNumbers(
    offset_dims=(), collapsed_slice_dims=(0,), start_index_map=(0,))


def _vgather(table, idx):
    return lax.gather(
        table, idx[:, None], _GATHER_DNUMS, (1,),
        mode=lax.GatherScatterMode.PROMISE_IN_BOUNDS)


def _silu_vec(xv, a_t, dy_t):
    v = jnp.maximum(xv * _SCALE + _BIAS, 0.0)
    idx = jnp.minimum(v, 15.5).astype(jnp.int32)
    return _vgather(a_t, idx) + v * _vgather(dy_t, idx)


def _compute_chunk(xb, ob, y0_t, dy_t):
    def vec_step(k, _):
        for u in range(UNROLL):
            j = (k * UNROLL + u) * L
            ob[pl.ds(j, L)] = _silu_vec(xb[pl.ds(j, L)], y0_t, dy_t)
        return 0

    lax.fori_loop(0, CHUNK // (L * UNROLL), vec_step, 0)


def _body(x_hbm, y0_hbm, dy_hbm, out_hbm,
          xb0, xb1, ob0, ob1, y0_v, dy_v, is0, is1, os0, os1):
    wid = lax.axis_index("s") * NC + lax.axis_index("c")
    base = wid * PER_W
    pltpu.sync_copy(y0_hbm, y0_v)
    pltpu.sync_copy(dy_hbm, dy_v)
    y0_t = y0_v[...]
    dy_t = dy_v[...]

    pltpu.async_copy(x_hbm.at[pl.ds(base, CHUNK)], xb0, is0)

    bufs = ((xb0, ob0, is0, os0), (xb1, ob1, is1, os1))

    def step(ci2, _):
        i0 = ci2 * 2
        for ph in (0, 1):
            xb, ob, isem, osem = bufs[ph]
            nxb, _, nisem, _ = bufs[1 - ph]
            i = i0 + ph
            nxt = i + 1

            if ph == 0:
                pltpu.async_copy(
                    x_hbm.at[pl.ds(base + nxt * CHUNK, CHUNK)], nxb, nisem)
            else:
                @pl.when(nxt < NCHUNK)
                def _():
                    pltpu.async_copy(
                        x_hbm.at[pl.ds(base + nxt * CHUNK, CHUNK)], nxb, nisem)

            pltpu.make_async_copy(
                x_hbm.at[pl.ds(base, CHUNK)], xb, isem).wait()

            @pl.when(i >= 2)
            def _():
                pltpu.make_async_copy(
                    ob, out_hbm.at[pl.ds(base, CHUNK)], osem).wait()

            _compute_chunk(xb, ob, y0_t, dy_t)
            pltpu.async_copy(
                ob, out_hbm.at[pl.ds(base + i * CHUNK, CHUNK)], osem)
        return 0

    lax.fori_loop(0, NCHUNK // 2, step, 0)
    pltpu.make_async_copy(ob0, out_hbm.at[pl.ds(base, CHUNK)], os0).wait()
    pltpu.make_async_copy(ob1, out_hbm.at[pl.ds(base, CHUNK)], os1).wait()


@functools.partial(
    pl.kernel,
    mesh=plsc.VectorSubcoreMesh(core_axis_name="c", subcore_axis_name="s"),
    out_type=jax.ShapeDtypeStruct((N,), jnp.float32),
    scratch_types=[
        pltpu.VMEM((CHUNK,), jnp.float32),
        pltpu.VMEM((CHUNK,), jnp.float32),
        pltpu.VMEM((CHUNK,), jnp.float32),
        pltpu.VMEM((CHUNK,), jnp.float32),
        pltpu.VMEM((16,), jnp.float32),
        pltpu.VMEM((16,), jnp.float32),
        pltpu.SemaphoreType.DMA,
        pltpu.SemaphoreType.DMA,
        pltpu.SemaphoreType.DMA,
        pltpu.SemaphoreType.DMA,
    ],
)
def _silu_sc(x_hbm, y0_hbm, dy_hbm, out_hbm,
             xb0, xb1, ob0, ob1, y0_v, dy_v, is0, is1, os0, os1):
    _body(x_hbm, y0_hbm, dy_hbm, out_hbm,
          xb0, xb1, ob0, ob1, y0_v, dy_v, is0, is1, os0, os1)


def kernel(x):
    return _silu_sc(x, jnp.asarray(_Y0), jnp.asarray(_DY))
